# Initial kernel scaffold; baseline (speedup 1.0000x reference)
import jax
import jax.numpy as jnp
from jax.experimental import pallas as pl


def kernel(x0, label, out_1, out_2, W1_l, b1, W1_r, W2_l, b2, W2_r, weight):
    raise NotImplementedError("write your pallas kernel here")


# speedup vs baseline: 8.2106x; 8.2106x over previous
import jax
import jax.numpy as jnp
from jax.experimental import pallas as pl


def _fused_sage_kernel(x_ref, w1_ref, b1_ref, w2_ref, b2_ref, w_ref, lab_ref,
                       loss_ref, preds_ref):
    x = x_ref[...]
    h1 = jax.lax.dot_general(x, w1_ref[...], (((1,), (1,)), ((), ())),
                             preferred_element_type=jnp.float32)
    h1 = jnp.maximum(h1 + b1_ref[...], 0.0)
    h2 = jax.lax.dot_general(h1, w2_ref[...], (((1,), (1,)), ((), ())),
                             preferred_element_type=jnp.float32)
    h2 = jnp.maximum(h2 + b2_ref[...], 0.0)
    s = jnp.dot(h2, w_ref[...], preferred_element_type=jnp.float32)

    B, C = s.shape
    m = jnp.max(s, axis=1, keepdims=True)
    lse = jnp.log(jnp.sum(jnp.exp(s - m), axis=1, keepdims=True)) + m
    iota = jax.lax.broadcasted_iota(jnp.int32, (B, C), 1)
    picked = jnp.sum(jnp.where(iota == lab_ref[...], s, 0.0),
                     axis=1, keepdims=True)
    loss_ref[...] = -jnp.sum(picked - lse, keepdims=True) / B
    preds_ref[...] = jnp.min(jnp.where(s == m, iota, C), axis=1, keepdims=True)


def kernel(x0, label, out_1, out_2, W1_l, b1, W1_r, W2_l, b2, W2_r, weight):
    B = x0.shape[0]
    F = x0.shape[-1]
    E = W1_r.shape[0]
    C = weight.shape[1]
    x = x0.reshape(-1, F)[:B]
    lab = label.astype(jnp.int32).reshape(B, 1)

    loss2d, preds2d = pl.pallas_call(
        _fused_sage_kernel,
        out_shape=(
            jax.ShapeDtypeStruct((1, 1), jnp.float32),
            jax.ShapeDtypeStruct((B, 1), jnp.int32),
        ),
    )(x, W1_r, b1.reshape(1, E), W2_r, b2.reshape(1, E), weight, lab)

    loss = loss2d.reshape(())
    preds = preds2d.reshape(B)
    return loss, preds, label.astype(jnp.int32)


# speedup vs baseline: 356.7576x; 43.4507x over previous
import jax
import jax.numpy as jnp
from jax.experimental import pallas as pl


def _fused_sage_kernel(x_ref, w1_ref, b1_ref, w2_ref, b2_ref, w_ref, lab_ref,
                       loss_ref, preds_ref):
    x = x_ref[...]
    h1 = jax.lax.dot_general(x, w1_ref[...], (((1,), (1,)), ((), ())),
                             preferred_element_type=jnp.float32)
    h1 = jnp.maximum(h1 + b1_ref[...], 0.0)
    h2 = jax.lax.dot_general(h1, w2_ref[...], (((1,), (1,)), ((), ())),
                             preferred_element_type=jnp.float32)
    h2 = jnp.maximum(h2 + b2_ref[...], 0.0)
    s = jnp.dot(h2, w_ref[...], preferred_element_type=jnp.float32)

    B, C = s.shape
    m = jnp.max(s, axis=1, keepdims=True)
    lse = jnp.log(jnp.sum(jnp.exp(s - m), axis=1, keepdims=True)) + m
    iota = jax.lax.broadcasted_iota(jnp.int32, (B, C), 1)
    picked = jnp.sum(jnp.where(iota == lab_ref[...], s, 0.0),
                     axis=1, keepdims=True)
    loss_ref[...] = -jnp.sum(picked - lse, keepdims=True) / B
    preds_ref[...] = jnp.min(jnp.where(s == m, iota, C), axis=1, keepdims=True)


def kernel(x0, label, out_1, out_2, W1_l, b1, W1_r, W2_l, b2, W2_r, weight):
    B = x0.shape[0]
    F = x0.shape[-1]
    E = W1_r.shape[0]
    C = weight.shape[1]
    nrows = -(-B // x0.shape[1])
    x = x0[:nrows].reshape(-1, F)[:B]
    lab = label.astype(jnp.int32).reshape(B, 1)

    loss2d, preds2d = pl.pallas_call(
        _fused_sage_kernel,
        out_shape=(
            jax.ShapeDtypeStruct((1, 1), jnp.float32),
            jax.ShapeDtypeStruct((B, 1), jnp.int32),
        ),
    )(x, W1_r, b1.reshape(1, E), W2_r, b2.reshape(1, E), weight, lab)

    loss = loss2d.reshape(())
    preds = preds2d.reshape(B)
    return loss, preds, label.astype(jnp.int32)


# speedup vs baseline: 576.0539x; 1.6147x over previous
import jax
import jax.numpy as jnp
from jax.experimental import pallas as pl


def _fused_sage_kernel(x_ref, w1_ref, b1_ref, w2_ref, b2_ref, w_ref, lab_ref,
                       loss_ref, preds_ref, labout_ref):
    x = x_ref[...]
    h1 = jax.lax.dot_general(x, w1_ref[...], (((1,), (1,)), ((), ())),
                             preferred_element_type=jnp.float32)
    h1 = jnp.maximum(h1 + b1_ref[...], 0.0)
    h2 = jax.lax.dot_general(h1, w2_ref[...], (((1,), (1,)), ((), ())),
                             preferred_element_type=jnp.float32)
    h2 = jnp.maximum(h2 + b2_ref[...], 0.0)
    sT = jax.lax.dot_general(w_ref[...], h2, (((0,), (1,)), ((), ())),
                             preferred_element_type=jnp.float32)

    C, B = sT.shape
    lab = lab_ref[...]
    m = jnp.max(sT, axis=0, keepdims=True)
    lse = jnp.log(jnp.sum(jnp.exp(sT - m), axis=0, keepdims=True)) + m
    iota = jax.lax.broadcasted_iota(jnp.int32, (C, B), 0)
    picked = jnp.sum(jnp.where(iota == lab, sT, 0.0),
                     axis=0, keepdims=True)
    loss_ref[...] = -jnp.sum(picked - lse, keepdims=True) / B
    preds_ref[...] = jnp.min(jnp.where(sT == m, iota, C), axis=0, keepdims=True)
    labout_ref[...] = lab


def kernel(x0, label, out_1, out_2, W1_l, b1, W1_r, W2_l, b2, W2_r, weight):
    B = x0.shape[0]
    F = x0.shape[-1]
    E = W1_r.shape[0]
    nrows = -(-B // x0.shape[1])
    x = x0[:nrows].reshape(-1, F)[:B]
    lab = label.astype(jnp.int32).reshape(1, B)

    loss2d, predsT, labT = pl.pallas_call(
        _fused_sage_kernel,
        out_shape=(
            jax.ShapeDtypeStruct((1, 1), jnp.float32),
            jax.ShapeDtypeStruct((1, B), jnp.int32),
            jax.ShapeDtypeStruct((1, B), jnp.int32),
        ),
    )(x, W1_r, b1.reshape(1, E), W2_r, b2.reshape(1, E), weight, lab)

    return loss2d.reshape(()), predsT.reshape(B), labT.reshape(B)
